# baseline (speedup 1.0000x reference)
import functools

import jax
import jax.numpy as jnp
from jax import lax
from jax.experimental import pallas as pl
from jax.experimental.pallas import tpu as pltpu
from jax.experimental.pallas import tpu_sc as plsc


def _make_sc_gather(V, D, B):
  info = plsc.get_sparse_core_info()
  NW = info.num_cores * info.num_subcores
  assert B % NW == 0
  per_w = B // NW
  C = 512
  while per_w % C:
    C //= 2
  steps = per_w // C
  mesh = plsc.VectorSubcoreMesh(core_axis_name="c", subcore_axis_name="s")

  @functools.partial(
      pl.kernel,
      mesh=mesh,
      out_type=jax.ShapeDtypeStruct((B, D), jnp.float32),
      scratch_types=[
          pltpu.VMEM((C,), jnp.int32),
          pltpu.VMEM((C, D), jnp.float32),
          pltpu.SemaphoreType.DMA,
      ],
  )
  def gather(idx_hbm, table_hbm, out_hbm, idx_v, rows_v, sem):
    wid = lax.axis_index("s") * info.num_cores + lax.axis_index("c")
    base = wid * per_w

    def body(t, carry):
      off = base + t * C
      pltpu.sync_copy(idx_hbm.at[pl.ds(off, C)], idx_v)
      pltpu.async_copy(table_hbm.at[idx_v], rows_v, sem).wait()
      pltpu.sync_copy(rows_v, out_hbm.at[pl.ds(off, C)])
      return carry

    lax.fori_loop(0, steps, body, 0)

  return gather


def _mlp1_body(e_ref, w_ref, b_ref, out_ref):
  h = jnp.dot(e_ref[...], w_ref[...], preferred_element_type=jnp.float32)
  out_ref[...] = jnp.maximum(h + b_ref[...], 0.0).astype(jnp.bfloat16)


def _lse_body(nv, bs, r_ref, w_ref, b_ref, lse_ref, m_ref, s_ref):
  j = pl.program_id(0)
  i = pl.program_id(1)
  sl = pl.ds(i * bs, bs)
  logits = jnp.dot(r_ref[...], w_ref[...], preferred_element_type=jnp.float32)
  logits = logits + b_ref[...]
  bm = jnp.max(logits, axis=1, keepdims=True)
  bsum = jnp.sum(jnp.exp(logits - bm), axis=1, keepdims=True)

  @pl.when(j == 0)
  def _init():
    m_ref[sl, :] = bm
    s_ref[sl, :] = bsum

  @pl.when(j > 0)
  def _update():
    m_old = m_ref[sl, :]
    m_new = jnp.maximum(m_old, bm)
    s_ref[sl, :] = s_ref[sl, :] * jnp.exp(m_old - m_new) + bsum * jnp.exp(bm - m_new)
    m_ref[sl, :] = m_new

  @pl.when(j == nv - 1)
  def _emit():
    lse_ref[...] = m_ref[sl, :] + jnp.log(s_ref[sl, :])


def _out_body(r_ref, w_ref, b_ref, lse_ref, out_ref):
  logits = jnp.dot(r_ref[...], w_ref[...], preferred_element_type=jnp.float32)
  out_ref[...] = logits + b_ref[...] - lse_ref[...]


def kernel(inputs, emb, W1, b1, W2, b2):
  Bt, ctx = inputs.shape
  V, D = emb.shape
  in_feat, H = W1.shape
  vocab = W2.shape[1]

  idx_flat = inputs.astype(jnp.int32).reshape(-1)
  gathered = _make_sc_gather(V, D, idx_flat.shape[0])(idx_flat, emb)
  embeds = gathered.reshape(Bt, ctx * D)

  BSA = 2048
  r = pl.pallas_call(
      _mlp1_body,
      grid=(Bt // BSA,),
      in_specs=[
          pl.BlockSpec((BSA, in_feat), lambda i: (i, 0)),
          pl.BlockSpec((in_feat, H), lambda i: (0, 0)),
          pl.BlockSpec((1, H), lambda i: (0, 0)),
      ],
      out_specs=pl.BlockSpec((BSA, H), lambda i: (i, 0)),
      out_shape=jax.ShapeDtypeStruct((Bt, H), jnp.bfloat16),
  )(embeds, W1, b1.reshape(1, H))

  VS = 2048
  NV = (vocab + VS - 1) // VS
  vpad = NV * VS - vocab
  W2p = jnp.pad(W2.astype(jnp.bfloat16), ((0, 0), (0, vpad)))
  b2p = jnp.pad(b2, (0, vpad), constant_values=-1e30).reshape(1, -1)

  BS = 2048
  NB = Bt // BS
  lse = pl.pallas_call(
      functools.partial(_lse_body, NV, BS),
      grid=(NV, NB),
      in_specs=[
          pl.BlockSpec((BS, H), lambda j, i: (i, 0)),
          pl.BlockSpec((H, VS), lambda j, i: (0, j)),
          pl.BlockSpec((1, VS), lambda j, i: (0, j)),
      ],
      out_specs=pl.BlockSpec((BS, 1), lambda j, i: (i, 0)),
      out_shape=jax.ShapeDtypeStruct((Bt, 1), jnp.float32),
      scratch_shapes=[
          pltpu.VMEM((Bt, 1), jnp.float32),
          pltpu.VMEM((Bt, 1), jnp.float32),
      ],
  )(r, W2p, b2p)

  out = pl.pallas_call(
      _out_body,
      grid=(NV, NB),
      in_specs=[
          pl.BlockSpec((BS, H), lambda j, i: (i, 0)),
          pl.BlockSpec((H, VS), lambda j, i: (0, j)),
          pl.BlockSpec((1, VS), lambda j, i: (0, j)),
          pl.BlockSpec((BS, 1), lambda j, i: (i, 0)),
      ],
      out_specs=pl.BlockSpec((BS, VS), lambda j, i: (i, j)),
      out_shape=jax.ShapeDtypeStruct((Bt, vocab), jnp.float32),
  )(r, W2p, b2p, lse)

  return out
